# Initial kernel scaffold; baseline (speedup 1.0000x reference)
import jax
import jax.numpy as jnp
from jax.experimental import pallas as pl


def kernel(x, condition, embed, embed_condition, Wc, bc, W2, b2, W3, b3, W4, b4):
    raise NotImplementedError("write your pallas kernel here")


# speedup vs baseline: 2.6852x; 2.6852x over previous
import functools

import jax
import jax.numpy as jnp
from jax import lax
from jax.experimental import pallas as pl
from jax.experimental.pallas import tpu as pltpu
from jax.experimental.pallas import tpu_sc as plsc

_B = 4096
_CH = 4
_L = 50
_H1 = 256
_H2 = 32
_OUT = 5
_LANES = 16
_NCHK = _H1 // _LANES
_SEG = _CH * _L


def _sc_gather_pool(x_flat, cond_flat, embed, embed_condition):
    mesh = plsc.VectorSubcoreMesh(
        core_axis_name="c", subcore_axis_name="s", num_cores=2, num_subcores=16
    )
    nw = mesh.num_cores * mesh.num_subcores
    spw = _B // nw
    n_chunks = 4
    spc = spw // n_chunks

    @functools.partial(
        pl.kernel,
        out_type=(
            jax.ShapeDtypeStruct((_CH * _B * _H1,), jnp.float32),
            jax.ShapeDtypeStruct((_B * _H1,), jnp.float32),
        ),
        mesh=mesh,
        scratch_types=[
            pltpu.VMEM((spc * _SEG,), jnp.int32),
            pltpu.VMEM((_SEG, _H1), jnp.float32),
            pltpu.VMEM((spw * _H1,), jnp.float32),
            pltpu.SemaphoreType.DMA,
        ],
    )
    def k(x_hbm, c_hbm, emb_hbm, embc_hbm, xs_out, cond_out,
          idx_v, rows_v, stage_v, sem):
        wid = lax.axis_index("s") * mesh.num_cores + lax.axis_index("c")
        b0 = wid * spw

        def gather200(table, goff):
            cp1 = pltpu.async_copy(
                table.at[idx_v.at[pl.ds(pl.multiple_of(goff, 8), 128)]],
                rows_v.at[pl.ds(0, 128)], sem)
            cp2 = pltpu.async_copy(
                table.at[idx_v.at[pl.ds(pl.multiple_of(goff + 128, 8),
                                        _SEG - 128)]],
                rows_v.at[pl.ds(128, _SEG - 128)], sem)
            cp1.wait()
            cp2.wait()

        def accum_rows(row0):
            init = tuple(jnp.zeros((_LANES,), jnp.float32)
                         for _ in range(_NCHK))

            @pl.loop(0, _L, init_carry=init)
            def acc_loop(j, acc):
                r = row0 + j
                return tuple(acc[c] + rows_v[r, pl.ds(c * _LANES, _LANES)]
                             for c in range(_NCHK))

            return acc_loop

        @pl.loop(0, n_chunks)
        def _chunk(ci):
            cb = b0 + ci * spc
            pltpu.sync_copy(
                x_hbm.at[pl.ds(pl.multiple_of(cb * _SEG, 8), spc * _SEG)],
                idx_v)

            @pl.loop(0, spc)
            def _grp(gi):
                gather200(emb_hbm, gi * _SEG)
                for ch in range(_CH):
                    acc = accum_rows(ch * _L)
                    soff = ch * (spc * _H1) + gi * _H1
                    for c in range(_NCHK):
                        stage_v[pl.ds(soff + c * _LANES, _LANES)] = acc[c]

            for ch in range(_CH):
                dst = pl.multiple_of((ch * _B + cb) * _H1, 8)
                pltpu.sync_copy(
                    stage_v.at[pl.ds(ch * spc * _H1, spc * _H1)],
                    xs_out.at[pl.ds(dst, spc * _H1)])

        pltpu.sync_copy(
            c_hbm.at[pl.ds(pl.multiple_of(b0 * _L, 8), spw * _L)], idx_v)

        @pl.loop(0, spw // 4)
        def _cgrp(gi):
            gather200(embc_hbm, gi * _SEG)
            for s in range(4):
                acc = accum_rows(s * _L)
                soff = (gi * 4 + s) * _H1
                for c in range(_NCHK):
                    stage_v[pl.ds(soff + c * _LANES, _LANES)] = acc[c]

        pltpu.sync_copy(
            stage_v, cond_out.at[pl.ds(pl.multiple_of(b0 * _H1, 8),
                                       spw * _H1)])

    return k(x_flat, cond_flat, embed, embed_condition)


def _tc_mlp(xs_raw, cond_raw, Wc, bc, W2, b2, W3, b3, W4, b4):
    bb = 512
    dn = (((1,), (1,)), ((), ()))

    def body(xs_ref, c_ref, wc_ref, bc_ref, w2_ref, b2_ref, w3_ref, b3_ref,
             w4_ref, b4_ref, out_ref):
        xs = jnp.maximum(xs_ref[...], 0.0)
        xsum = xs[0] + xs[1] + xs[2] + xs[3]
        c = jnp.maximum(c_ref[...], 0.0) + xsum
        c2 = lax.dot_general(c, wc_ref[...], dn,
                             preferred_element_type=jnp.float32)
        c2 = jnp.maximum(c2 + bc_ref[...], 0.0)
        for ch in range(_CH):
            h = lax.dot_general(xs[ch], w2_ref[...], dn,
                                preferred_element_type=jnp.float32)
            h = jnp.maximum(h + b2_ref[...] + c2, 0.0)
            h = lax.dot_general(h, w3_ref[...], dn,
                                preferred_element_type=jnp.float32)
            h = jnp.maximum(h + b3_ref[...], 0.0)
            o = lax.dot_general(h, w4_ref[...], dn,
                                preferred_element_type=jnp.float32)
            out_ref[ch] = o + b4_ref[...]

    full = lambda shape: pl.BlockSpec(shape, lambda i: tuple(0 for _ in shape))
    return pl.pallas_call(
        body,
        grid=(_B // bb,),
        in_specs=[
            pl.BlockSpec((_CH, bb, _H1), lambda i: (0, i, 0)),
            pl.BlockSpec((bb, _H1), lambda i: (i, 0)),
            full((_H2, _H1)),
            full((1, _H2)),
            full((_H2, _H1)),
            full((1, _H2)),
            full((_H2, _H2)),
            full((1, _H2)),
            full((_OUT, _H2)),
            full((1, _OUT)),
        ],
        out_specs=pl.BlockSpec((_CH, bb, _OUT), lambda i: (0, i, 0)),
        out_shape=jax.ShapeDtypeStruct((_CH, _B, _OUT), jnp.float32),
    )(xs_raw, cond_raw, Wc, bc.reshape(1, _H2), W2, b2.reshape(1, _H2),
      W3, b3.reshape(1, _H2), W4, b4.reshape(1, _OUT))


def kernel(x, condition, embed, embed_condition, Wc, bc, W2, b2, W3, b3,
           W4, b4):
    x_flat = x.reshape(-1).astype(jnp.int32)
    cond_flat = condition.reshape(-1).astype(jnp.int32)
    xs_flat, cond_pool = _sc_gather_pool(x_flat, cond_flat, embed,
                                         embed_condition)
    xs_raw = xs_flat.reshape(_CH, _B, _H1)
    cond_raw = cond_pool.reshape(_B, _H1)
    out = _tc_mlp(xs_raw, cond_raw, Wc, bc, W2, b2, W3, b3, W4, b4)
    return out.transpose(1, 0, 2)


# speedup vs baseline: 4.2253x; 1.5736x over previous
import functools

import jax
import jax.numpy as jnp
from jax import lax
from jax.experimental import pallas as pl
from jax.experimental.pallas import tpu as pltpu
from jax.experimental.pallas import tpu_sc as plsc

_B = 4096
_CH = 4
_L = 50
_H1 = 256
_H2 = 32
_OUT = 5
_LANES = 16
_NCHK = _H1 // _LANES
_SEG = _CH * _L


def _sc_gather_pool(x_flat, cond_flat, embed, embed_condition):
    mesh = plsc.VectorSubcoreMesh(
        core_axis_name="c", subcore_axis_name="s", num_cores=2, num_subcores=16
    )
    nw = mesh.num_cores * mesh.num_subcores
    spw = _B // nw
    gpc = 16
    idx_words = gpc * _SEG
    stage_words = gpc * _CH * _H1

    @functools.partial(
        pl.kernel,
        out_type=(
            jax.ShapeDtypeStruct((_CH * _B * _H1,), jnp.float32),
            jax.ShapeDtypeStruct((_B * _H1,), jnp.float32),
        ),
        mesh=mesh,
        scratch_types=[
            pltpu.VMEM((idx_words,), jnp.int32),
            pltpu.VMEM((_SEG, _H1), jnp.float32),
            pltpu.VMEM((_SEG, _H1), jnp.float32),
            pltpu.VMEM((stage_words,), jnp.float32),
            pltpu.SemaphoreType.DMA,
            pltpu.SemaphoreType.DMA,
        ],
    )
    def k(x_hbm, c_hbm, emb_hbm, embc_hbm, xs_out, cond_out,
          idx_v, buf_a, buf_b, stage_v, sem_a, sem_b):
        wid = lax.axis_index("s") * mesh.num_cores + lax.axis_index("c")
        b0 = wid * spw

        def gstart(table, goff, buf, sem):
            pltpu.async_copy(
                table.at[idx_v.at[pl.ds(pl.multiple_of(goff, 8), 128)]],
                buf.at[pl.ds(0, 128)], sem)
            pltpu.async_copy(
                table.at[idx_v.at[pl.ds(pl.multiple_of(goff + 128, 8),
                                        _SEG - 128)]],
                buf.at[pl.ds(128, _SEG - 128)], sem)

        def gwait(table, buf, sem):
            pltpu.make_async_copy(
                table.at[idx_v.at[pl.ds(0, 128)]],
                buf.at[pl.ds(0, 128)], sem).wait()
            pltpu.make_async_copy(
                table.at[idx_v.at[pl.ds(128, _SEG - 128)]],
                buf.at[pl.ds(128, _SEG - 128)], sem).wait()

        def accum(buf, g, stage_off):
            for s in range(4):
                init = tuple(jnp.zeros((_LANES,), jnp.float32)
                             for _ in range(_NCHK))

                @pl.loop(0, _L, init_carry=init, unroll=2)
                def acc_loop(j, acc):
                    r = s * _L + j
                    return tuple(
                        acc[c] + buf[r, pl.ds(c * _LANES, _LANES)]
                        for c in range(_NCHK))

                off = stage_off(g, s)
                for c in range(_NCHK):
                    stage_v[pl.ds(off + c * _LANES, _LANES)] = acc_loop[c]

        def run_phase(table, idx_hbm, n_chunks, idx_off, stage_off, write_fn):
            @pl.loop(0, n_chunks)
            def _chunk(ci):
                pltpu.sync_copy(
                    idx_hbm.at[pl.ds(pl.multiple_of(idx_off(ci), 8),
                                     idx_words)], idx_v)
                gstart(table, 0, buf_a, sem_a)

                @pl.loop(0, gpc, step=2)
                def _g(g):
                    gstart(table, (g + 1) * _SEG, buf_b, sem_b)
                    gwait(table, buf_a, sem_a)
                    accum(buf_a, g, stage_off)

                    @pl.when(g < gpc - 2)
                    def _():
                        gstart(table, (g + 2) * _SEG, buf_a, sem_a)

                    gwait(table, buf_b, sem_b)
                    accum(buf_b, g + 1, stage_off)

                write_fn(ci)

        def xs_write(ci):
            cb = b0 + ci * gpc
            for ch in range(_CH):
                pltpu.sync_copy(
                    stage_v.at[pl.ds(ch * gpc * _H1, gpc * _H1)],
                    xs_out.at[pl.ds(pl.multiple_of((ch * _B + cb) * _H1, 8),
                                    gpc * _H1)])

        run_phase(
            emb_hbm, x_hbm, spw // gpc,
            lambda ci: (b0 + ci * gpc) * _SEG,
            lambda g, s: s * (gpc * _H1) + g * _H1,
            xs_write)

        def cond_write(ci):
            cb = b0 + ci * 4 * gpc
            pltpu.sync_copy(
                stage_v,
                cond_out.at[pl.ds(pl.multiple_of(cb * _H1, 8), stage_words)])

        run_phase(
            embc_hbm, c_hbm, spw // (4 * gpc),
            lambda ci: (b0 + ci * 4 * gpc) * _L,
            lambda g, s: (g * 4 + s) * _H1,
            cond_write)

    return k(x_flat, cond_flat, embed, embed_condition)


def _tc_mlp(xs_raw, cond_raw, Wc, bc, W2, b2, W3, b3, W4, b4):
    bb = 512
    dn = (((1,), (1,)), ((), ()))

    def body(xs_ref, c_ref, wc_ref, bc_ref, w2_ref, b2_ref, w3_ref, b3_ref,
             w4_ref, b4_ref, out_ref):
        xs = jnp.maximum(xs_ref[...], 0.0)
        xsum = xs[0] + xs[1] + xs[2] + xs[3]
        c = jnp.maximum(c_ref[...], 0.0) + xsum
        c2 = lax.dot_general(c, wc_ref[...], dn,
                             preferred_element_type=jnp.float32)
        c2 = jnp.maximum(c2 + bc_ref[...], 0.0)
        for ch in range(_CH):
            h = lax.dot_general(xs[ch], w2_ref[...], dn,
                                preferred_element_type=jnp.float32)
            h = jnp.maximum(h + b2_ref[...] + c2, 0.0)
            h = lax.dot_general(h, w3_ref[...], dn,
                                preferred_element_type=jnp.float32)
            h = jnp.maximum(h + b3_ref[...], 0.0)
            o = lax.dot_general(h, w4_ref[...], dn,
                                preferred_element_type=jnp.float32)
            out_ref[ch] = o + b4_ref[...]

    full = lambda shape: pl.BlockSpec(shape, lambda i: tuple(0 for _ in shape))
    return pl.pallas_call(
        body,
        grid=(_B // bb,),
        in_specs=[
            pl.BlockSpec((_CH, bb, _H1), lambda i: (0, i, 0)),
            pl.BlockSpec((bb, _H1), lambda i: (i, 0)),
            full((_H2, _H1)),
            full((1, _H2)),
            full((_H2, _H1)),
            full((1, _H2)),
            full((_H2, _H2)),
            full((1, _H2)),
            full((_OUT, _H2)),
            full((1, _OUT)),
        ],
        out_specs=pl.BlockSpec((_CH, bb, _OUT), lambda i: (0, i, 0)),
        out_shape=jax.ShapeDtypeStruct((_CH, _B, _OUT), jnp.float32),
    )(xs_raw, cond_raw, Wc, bc.reshape(1, _H2), W2, b2.reshape(1, _H2),
      W3, b3.reshape(1, _H2), W4, b4.reshape(1, _OUT))


def kernel(x, condition, embed, embed_condition, Wc, bc, W2, b2, W3, b3,
           W4, b4):
    x_flat = x.reshape(-1).astype(jnp.int32)
    cond_flat = condition.reshape(-1).astype(jnp.int32)
    xs_flat, cond_pool = _sc_gather_pool(x_flat, cond_flat, embed,
                                         embed_condition)
    xs_raw = xs_flat.reshape(_CH, _B, _H1)
    cond_raw = cond_pool.reshape(_B, _H1)
    out = _tc_mlp(xs_raw, cond_raw, Wc, bc, W2, b2, W3, b3, W4, b4)
    return out.transpose(1, 0, 2)
